# Initial kernel scaffold; baseline (speedup 1.0000x reference)
import jax
import jax.numpy as jnp
from jax.experimental import pallas as pl


def kernel(x, y, labels, W_in, b_in, Wq, Wk, Wv, Wo, W1, W2, W_out, b_out):
    raise NotImplementedError("write your pallas kernel here")


# speedup vs baseline: 2.4590x; 2.4590x over previous
import functools

import numpy as np
import jax
import jax.numpy as jnp
from jax import lax
from jax.experimental import pallas as pl
from jax.experimental.pallas import tpu as pltpu
from jax.experimental.pallas import tpu_sc as plsc

_ENC_HALF = 16
_N_SERIES = 33
_NC = 32 * 120 + 20
_NT = 10
_DM = 64
_NH, _HD = 4, 4
_PROJ = _NH * _HD
_MB = 2


_TPAD = 5824
_PW = 48


def _sc_pack_body(xy_hbm, pk_hbm, inb, outb):
    wid = lax.axis_index("s") * 2 + lax.axis_index("c")
    b = lax.rem(wid, 16)
    a = wid // 16
    i16 = lax.iota(jnp.int32, 16)

    pltpu.sync_copy(xy_hbm.at[a, b], inb)
    for j in range(120):
        outb[0, pl.ds(48 * j, 16)] = plsc.load_gather(inb, [i16 * 180 + j])
        outb[0, pl.ds(48 * j + 16, 16)] = plsc.load_gather(
            inb, [i16 * 180 + (2880 + j)])
        outb[0, pl.ds(48 * j + 32, 16)] = plsc.load_gather(
            inb, [i16 * 0 + (5760 + min(j, 60))])
    pltpu.sync_copy(outb, pk_hbm.at[a, b])


def _sc_pack(x2, y2):
    B = x2.shape[0]
    xy = jnp.stack([
        jnp.pad(x2, ((0, 0), (0, _TPAD - x2.shape[1]))),
        jnp.pad(y2, ((0, 0), (0, _TPAD - y2.shape[1]))),
    ])
    fn = functools.partial(
        pl.kernel,
        mesh=plsc.VectorSubcoreMesh(core_axis_name="c", subcore_axis_name="s"),
        out_type=jax.ShapeDtypeStruct((2, B, 1, 120 * _PW), jnp.float32),
        scratch_types=[
            pltpu.VMEM((_TPAD,), jnp.float32),
            pltpu.VMEM((1, 120 * _PW), jnp.float32),
        ],
        compiler_params=pltpu.CompilerParams(needs_layout_passes=False),
    )(_sc_pack_body)
    pk = fn(xy).reshape(2, B, 120, _PW)
    return pk[0], pk[1]


def _constants():
    k = jax.lax.broadcasted_iota(
        jnp.int32, (1, _ENC_HALF), 1).astype(jnp.float32)
    freq = (2.0 * np.pi / 0.1) * jnp.exp(-(k / 15.0) * np.log(20.0))
    r40 = jax.lax.broadcasted_iota(jnp.int32, (40, _PROJ), 0) // _NT
    c40 = jax.lax.broadcasted_iota(jnp.int32, (40, _PROJ), 1) // _HD
    hm = (r40 == c40).astype(jnp.float32)
    c16 = jax.lax.broadcasted_iota(jnp.int32, (1, _PROJ), 1) // _HD
    hm_h = [(c16 == h).astype(jnp.float32) for h in range(_NH)]
    return freq, hm, hm_h


def _fwd(xp_ref, yp_ref, lab_ref, Win_ref, bin_ref, Wq_ref, Wk_ref, Wv_ref,
         Wo_ref, W1_ref, W2_ref, Wout_ref, bout_ref, o_ref,
         *scratch):
    _FREQ, _HM, _HM_H = _constants()

    Win = Win_ref[...]
    Ws = Win[0:16]
    Wc = Win[16:32]
    w_y = Win[65:66]
    w_yd = Win[66:67] + Win[69:70]
    w_xd = Win[67:68] + Win[68:69]
    w_xn = Win[70:71]
    w_yn = Win[71:72]
    bias = bin_ref[0]

    labs = lab_ref[0]
    jrow = jax.lax.broadcasted_iota(jnp.int32, (_N_SERIES, _N_SERIES), 0)
    M = (labs == jrow).astype(jnp.float32)
    Lrows = jax.lax.dot_general(M, Win[32:65], (((0,), (0,)), ((), ())),
                                preferred_element_type=jnp.float32)

    lane = jax.lax.broadcasted_iota(jnp.int32, (1, 128), 1)
    flane = (2.0 * np.pi / 0.1) * jnp.exp(
        -((lane % _ENC_HALF).astype(jnp.float32) / 15.0) * np.log(20.0))
    sub8 = jax.lax.broadcasted_iota(jnp.int32, (8, 128), 0)
    E = jnp.where(sub8 == lane // _ENC_HALF, flane, 0.0)

    WkT = jnp.concatenate(
        [jnp.transpose(Wk_ref[l]) for l in range(3)], axis=0)
    Wv_all = jnp.concatenate([Wv_ref[l] for l in range(3)], axis=1)

    def features(bb):
        h_ref = scratch[bb]
        X = xp_ref[bb][:, 0:33]
        Y = yp_ref[bb][:, 0:33]

        XD = X - X[0:1, :]
        YD = Y - Y[0:1, :]
        inv_dx = 1.0 / (jnp.max(jnp.abs(XD), axis=0, keepdims=True) + 1e-6)
        inv_dy = 1.0 / (jnp.max(jnp.abs(YD), axis=0, keepdims=True) + 1e-6)

        for g in range(4):
            ANG = jnp.dot(X[:, 8 * g:8 * g + 8], E,
                          preferred_element_type=jnp.float32)
            SN = jnp.sin(ANG)
            CS = jnp.cos(ANG)
            for m in range(8):
                i = 8 * g + m
                c = slice(i, i + 1)
                ux = w_xd + inv_dx[0:1, c] * w_xn
                uy = w_yd + inv_dy[0:1, c] * w_yn
                rowc = (Lrows[i:i + 1] + bias
                        - X[0:1, c] * ux - Y[0:1, c] * uy)
                h = (jnp.dot(SN[:, 16 * m:16 * m + 16], Ws,
                             preferred_element_type=jnp.float32)
                     + jnp.dot(CS[:, 16 * m:16 * m + 16], Wc,
                               preferred_element_type=jnp.float32)
                     + X[:, c] * ux + Y[:, c] * (w_y + uy) + rowc)
                h_ref[pl.ds(120 * i, 120), :] = h

        xs = X[0:30, 32:33]
        ys = Y[0:30, 32:33]
        x0 = xs[0:1]
        y0 = ys[0:1]
        ivx = 1.0 / (jnp.max(jnp.abs(xs - x0), axis=0, keepdims=True) + 1e-6)
        ivy = 1.0 / (jnp.max(jnp.abs(ys - y0), axis=0, keepdims=True) + 1e-6)
        ux = w_xd + ivx * w_xn
        uy = w_yd + ivy * w_yn
        ang = xs * _FREQ
        h_x = (jnp.dot(jnp.sin(ang), Ws, preferred_element_type=jnp.float32)
               + jnp.dot(jnp.cos(ang), Wc, preferred_element_type=jnp.float32)
               + xs * ux + (Lrows[32:33] + bias - x0 * ux))
        h_full = h_x + ys * (w_y + uy) - y0 * uy
        h_ref[pl.ds(3840, 20), :] = h_full[0:20]
        h_t = h_x[20:30]

        Hv = h_ref[...]
        HT = jnp.transpose(Hv)
        KT = jnp.dot(WkT, HT, preferred_element_type=jnp.float32)
        V = jnp.dot(Hv, Wv_all, preferred_element_type=jnp.float32)
        return h_t, KT, V

    def layer(l, h_t, KT, V):
        q = jnp.dot(h_t, Wq_ref[l], preferred_element_type=jnp.float32)
        qe = jnp.concatenate([q, q, q, q], axis=0) * (_HM * 0.5)
        S = jnp.dot(qe, KT[16 * l:16 * l + 16],
                    preferred_element_type=jnp.float32)
        P = jnp.exp(S - jnp.max(S, axis=1, keepdims=True))
        den = jnp.sum(P, axis=1, keepdims=True)
        oe = jnp.dot(P, V[:, 16 * l:16 * l + 16],
                     preferred_element_type=jnp.float32)
        onum = (oe[0:10] * _HM_H[0] + oe[10:20] * _HM_H[1]
                + oe[20:30] * _HM_H[2] + oe[30:40] * _HM_H[3])
        oden = (den[0:10] * _HM_H[0] + den[10:20] * _HM_H[1]
                + den[20:30] * _HM_H[2] + den[30:40] * _HM_H[3])
        o = onum / oden
        h_t = h_t + jnp.dot(o, Wo_ref[l], preferred_element_type=jnp.float32)
        ff = jnp.maximum(jnp.dot(h_t, W1_ref[l],
                                 preferred_element_type=jnp.float32), 0.0)
        return h_t + jnp.dot(ff, W2_ref[l], preferred_element_type=jnp.float32)

    state = [features(bb) for bb in range(_MB)]
    for l in range(3):
        state = [(layer(l, h_t, KT, V), KT, V) for (h_t, KT, V) in state]

    for bb in range(_MB):
        h_t = state[bb][0]
        out = jnp.dot(h_t, Wout_ref[...],
                      preferred_element_type=jnp.float32) + bout_ref[0]
        mean = out[:, 0:1]
        sigma = jax.nn.softplus(out[:, 1:2])
        o_ref[bb] = jnp.concatenate([mean, sigma], axis=1)


def kernel(x, y, labels, W_in, b_in, Wq, Wk, Wv, Wo, W1, W2, W_out, b_out):
    B, T = x.shape[0], x.shape[1]
    xp, yp = _sc_pack(x.reshape(B, T), y.reshape(B, T))
    lab3 = labels.reshape(1, 1, _N_SERIES)
    bin3 = b_in.reshape(1, 1, _DM)
    bout3 = b_out.reshape(1, 1, 2)

    in_specs = [
        pl.BlockSpec((_MB, 120, _PW), lambda b: (b, 0, 0)),
        pl.BlockSpec((_MB, 120, _PW), lambda b: (b, 0, 0)),
        pl.BlockSpec((1, 1, _N_SERIES), lambda b: (0, 0, 0)),
        pl.BlockSpec(W_in.shape, lambda b: (0, 0)),
        pl.BlockSpec((1, 1, _DM), lambda b: (0, 0, 0)),
        pl.BlockSpec(Wq.shape, lambda b: (0, 0, 0)),
        pl.BlockSpec(Wk.shape, lambda b: (0, 0, 0)),
        pl.BlockSpec(Wv.shape, lambda b: (0, 0, 0)),
        pl.BlockSpec(Wo.shape, lambda b: (0, 0, 0)),
        pl.BlockSpec(W1.shape, lambda b: (0, 0, 0)),
        pl.BlockSpec(W2.shape, lambda b: (0, 0, 0)),
        pl.BlockSpec(W_out.shape, lambda b: (0, 0)),
        pl.BlockSpec((1, 1, 2), lambda b: (0, 0, 0)),
    ]
    out = pl.pallas_call(
        _fwd,
        grid=(B // _MB,),
        in_specs=in_specs,
        out_specs=pl.BlockSpec((_MB, _NT, 2), lambda b: (b, 0, 0)),
        out_shape=jax.ShapeDtypeStruct((B, _NT, 2), jnp.float32),
        scratch_shapes=[
            pltpu.VMEM((_NC, _DM), jnp.float32),
        ] * _MB,
        compiler_params=pltpu.CompilerParams(
            dimension_semantics=("parallel",)),
    )(xp, yp, lab3, W_in, bin3, Wq, Wk, Wv, Wo, W1, W2, W_out, bout3)
    return out
